# Initial kernel scaffold; baseline (speedup 1.0000x reference)
import jax
import jax.numpy as jnp
from jax.experimental import pallas as pl


def kernel(x, pos_table):
    raise NotImplementedError("write your pallas kernel here")


# speedup vs baseline: 1.7205x; 1.7205x over previous
import functools

import jax
import jax.numpy as jnp
from jax.experimental import pallas as pl


def _add_block(x_ref, pos_ref, o_ref):
    o_ref[...] = x_ref[...] + pos_ref[...][None, :, :]


@functools.partial(jax.jit, static_argnames=())
def kernel(x, pos_table):
    batch, seq_len, d = x.shape
    s_blk = 256
    grid = (seq_len // s_blk,)
    return pl.pallas_call(
        _add_block,
        grid=grid,
        in_specs=[
            pl.BlockSpec((batch, s_blk, d), lambda s: (0, s, 0)),
            pl.BlockSpec((s_blk, d), lambda s: (s, 0)),
        ],
        out_specs=pl.BlockSpec((batch, s_blk, d), lambda s: (0, s, 0)),
        out_shape=jax.ShapeDtypeStruct((batch, seq_len, d), x.dtype),
    )(x, pos_table[:seq_len])


# speedup vs baseline: 1.7231x; 1.0015x over previous
import functools

import jax
import jax.numpy as jnp
from jax.experimental import pallas as pl


def _add_block(x_ref, pos_ref, o_ref):
    o_ref[...] = x_ref[...] + pos_ref[...][None, :, :]


@functools.partial(jax.jit, static_argnames=())
def kernel(x, pos_table):
    batch, seq_len, d = x.shape
    s_blk = 512
    grid = (seq_len // s_blk,)
    return pl.pallas_call(
        _add_block,
        grid=grid,
        in_specs=[
            pl.BlockSpec((batch, s_blk, d), lambda s: (0, s, 0)),
            pl.BlockSpec((s_blk, d), lambda s: (s, 0)),
        ],
        out_specs=pl.BlockSpec((batch, s_blk, d), lambda s: (0, s, 0)),
        out_shape=jax.ShapeDtypeStruct((batch, seq_len, d), x.dtype),
    )(x, pos_table[:seq_len])
